# baseline (speedup 1.0000x reference)
import functools

import jax
import jax.numpy as jnp
from jax import lax
from jax.experimental import pallas as pl
from jax.experimental.pallas import tpu as pltpu, tpu_sc as plsc

D_MODEL = 64
SCALE = 8.0
VOCAB = 1000000

_info = plsc.get_sparse_core_info()
_NC, _NS, _L = _info.num_cores, _info.num_subcores, _info.num_lanes
_NW = _NC * _NS

_C = 256
_IDX_W = 128
_G = _C // _IDX_W
_NBUF = 4
_LA = 2

_BT = 2048
_K = 512000
_NBLK = _K // _BT
_TBLK_MAX = (VOCAB + _BT - 1) // _BT - 1


def _transpose_scale(table_t):
    def body(a_ref, b_ref, o_ref):
        o_ref[...] = jnp.concatenate(
            [a_ref[...].T, b_ref[...].T], axis=1) * SCALE

    return pl.pallas_call(
        body,
        grid=(_NBLK,),
        in_specs=[
            pl.BlockSpec((D_MODEL, _BT), lambda j: (0, j)),
            pl.BlockSpec(
                (D_MODEL, _BT),
                lambda j: (0, jnp.minimum(j + _NBLK, _TBLK_MAX)),
            ),
        ],
        out_specs=pl.BlockSpec((_BT, 128), lambda j: (j, 0)),
        out_shape=jax.ShapeDtypeStruct((_K, 128), jnp.float32),
    )(table_t, table_t)


def _make_lookup(B: int):
    assert B % (_NW * _C) == 0
    b_per_w = B // _NW
    n_chunks = b_per_w // _C
    assert n_chunks % _NBUF == 0 and n_chunks > _NBUF
    mesh = plsc.VectorSubcoreMesh(core_axis_name="c", subcore_axis_name="s")

    @functools.partial(
        pl.kernel,
        mesh=mesh,
        compiler_params=pltpu.CompilerParams(use_tc_tiling_on_sc=False),
        out_type=jax.ShapeDtypeStruct((B, 128), jnp.float32),
        scratch_types=[
            pltpu.VMEM((b_per_w,), jnp.int32),
            pltpu.VMEM((_NBUF, _C, D_MODEL), jnp.float32),
            pltpu.SemaphoreType.DMA((_NBUF,)),
            pltpu.SemaphoreType.DMA((_NBUF,)),
        ],
    )
    def lookup(idx_hbm, table_hbm, out_hbm, idx_v, rows_v, semg, semo):
        wid = lax.axis_index("s") * _NC + lax.axis_index("c")
        base = wid * b_per_w

        pltpu.sync_copy(idx_hbm.at[pl.ds(base, b_per_w)], idx_v)

        @plsc.parallel_loop(0, b_per_w // _L, unroll=8)
        def _remap(i):
            sl = pl.ds(i * _L, _L)
            v = idx_v[sl]
            idx_v[sl] = v * 2 - jnp.where(v >= _K, 2 * _K - 1, 0)

        def gather_descs(h, b):
            return [
                pltpu.make_async_copy(
                    table_hbm.at[idx_v.at[pl.ds(h * _C + j * _IDX_W, _IDX_W)]],
                    rows_v.at[b, pl.ds(j * _IDX_W, _IDX_W)],
                    semg.at[b],
                )
                for j in range(_G)
            ]

        def out_desc(g, b):
            return pltpu.make_async_copy(
                rows_v.at[b],
                out_hbm.at[pl.ds(base + g * _C, _C), pl.ds(0, D_MODEL)],
                semo.at[b],
            )

        def fire_gather(h, b):
            for d in gather_descs(h, b):
                d.start()

        for h in range(_LA):
            fire_gather(h, h % _NBUF)

        @pl.loop(0, n_chunks, step=_NBUF)
        def _chunk_group(go):
            for b in range(_NBUF):
                g = go + b
                for d in gather_descs(g, b):
                    d.wait()
                out_desc(g, b).start()
                bh = (b + _LA) % _NBUF

                @pl.when(jnp.logical_and(g + _LA < n_chunks,
                                         g + _LA - _NBUF >= 0))
                def _wait_old_writeback():
                    out_desc(g + _LA - _NBUF, bh).wait()

                @pl.when(g + _LA < n_chunks)
                def _fire_next():
                    fire_gather(g + _LA, bh)

        for k in range(_NBUF):
            g = n_chunks - _NBUF + k
            out_desc(g, g % _NBUF).wait()

    return lookup


_lookup_819200 = _make_lookup(4096 * 200)


def kernel(x, table):
    flat = x.reshape(-1).astype(jnp.int32)
    scaled_wide = _transpose_scale(table.T)
    scaled_rows = scaled_wide.reshape(2 * _K, D_MODEL)
    out = _lookup_819200(flat, scaled_rows)
    return out[:, :D_MODEL].reshape(x.shape + (D_MODEL,))


# speedup vs baseline: 1.1597x; 1.1597x over previous
import functools

import jax
import jax.numpy as jnp
from jax import lax
from jax.experimental import pallas as pl
from jax.experimental.pallas import tpu as pltpu, tpu_sc as plsc

D_MODEL = 64
SCALE = 8.0
VOCAB = 1000000

_info = plsc.get_sparse_core_info()
_NC, _NS, _L = _info.num_cores, _info.num_subcores, _info.num_lanes
_NW = _NC * _NS

_C = 256
_IDX_W = 128
_G = _C // _IDX_W
_NBUF = 4
_LA = 2

_BT = 6400
_K = 512000
_NBLK = _K // _BT
_TBLK_MAX = (VOCAB + _BT - 1) // _BT - 1


def _transpose_scale(table_t):
    def body(a_ref, b_ref, o_ref):
        o_ref[...] = jnp.concatenate(
            [a_ref[...].T, b_ref[...].T], axis=1) * SCALE

    return pl.pallas_call(
        body,
        grid=(_NBLK,),
        in_specs=[
            pl.BlockSpec((D_MODEL, _BT), lambda j: (0, j)),
            pl.BlockSpec(
                (D_MODEL, _BT),
                lambda j: (0, jnp.minimum(j + _NBLK, _TBLK_MAX)),
            ),
        ],
        out_specs=pl.BlockSpec((_BT, 128), lambda j: (j, 0)),
        out_shape=jax.ShapeDtypeStruct((_K, 128), jnp.float32),
    )(table_t, table_t)


def _make_lookup(B: int):
    assert B % (_NW * _C) == 0
    b_per_w = B // _NW
    n_chunks = b_per_w // _C
    assert n_chunks % _NBUF == 0 and n_chunks > _NBUF
    mesh = plsc.VectorSubcoreMesh(core_axis_name="c", subcore_axis_name="s")

    @functools.partial(
        pl.kernel,
        mesh=mesh,
        compiler_params=pltpu.CompilerParams(use_tc_tiling_on_sc=False),
        out_type=jax.ShapeDtypeStruct((B, 128), jnp.float32),
        scratch_types=[
            pltpu.VMEM((b_per_w,), jnp.int32),
            pltpu.VMEM((_NBUF, _C, D_MODEL), jnp.float32),
            pltpu.SemaphoreType.DMA((_NBUF,)),
            pltpu.SemaphoreType.DMA((_NBUF,)),
        ],
    )
    def lookup(idx_hbm, table_hbm, out_hbm, idx_v, rows_v, semg, semo):
        wid = lax.axis_index("s") * _NC + lax.axis_index("c")
        base = wid * b_per_w

        pltpu.sync_copy(idx_hbm.at[pl.ds(base, b_per_w)], idx_v)

        @plsc.parallel_loop(0, b_per_w // _L, unroll=8)
        def _remap(i):
            sl = pl.ds(i * _L, _L)
            v = idx_v[sl]
            idx_v[sl] = v * 2 - jnp.where(v >= _K, 2 * _K - 1, 0)

        def gather_descs(h, b):
            return [
                pltpu.make_async_copy(
                    table_hbm.at[idx_v.at[pl.ds(h * _C + j * _IDX_W, _IDX_W)]],
                    rows_v.at[b, pl.ds(j * _IDX_W, _IDX_W)],
                    semg.at[b],
                )
                for j in range(_G)
            ]

        def out_desc(g, b):
            return pltpu.make_async_copy(
                rows_v.at[b],
                out_hbm.at[pl.ds(base + g * _C, _C), pl.ds(0, D_MODEL)],
                semo.at[b],
            )

        def fire_gather(h, b):
            for d in gather_descs(h, b):
                d.start()

        for h in range(_LA):
            fire_gather(h, h % _NBUF)

        @pl.loop(0, n_chunks, step=_NBUF)
        def _chunk_group(go):
            for b in range(_NBUF):
                g = go + b
                for d in gather_descs(g, b):
                    d.wait()
                out_desc(g, b).start()
                bh = (b + _LA) % _NBUF

                @pl.when(jnp.logical_and(g + _LA < n_chunks,
                                         g + _LA - _NBUF >= 0))
                def _wait_old_writeback():
                    out_desc(g + _LA - _NBUF, bh).wait()

                @pl.when(g + _LA < n_chunks)
                def _fire_next():
                    fire_gather(g + _LA, bh)

        for k in range(_NBUF):
            g = n_chunks - _NBUF + k
            out_desc(g, g % _NBUF).wait()

    return lookup


_lookup_819200 = _make_lookup(4096 * 200)


def kernel(x, table):
    flat = x.reshape(-1).astype(jnp.int32)
    scaled_wide = _transpose_scale(table.T)
    scaled_rows = scaled_wide.reshape(2 * _K, D_MODEL)
    out = _lookup_819200(flat, scaled_rows)
    return out[:, :D_MODEL].reshape(x.shape + (D_MODEL,))


# speedup vs baseline: 1.2069x; 1.0406x over previous
import functools

import jax
import jax.numpy as jnp
from jax import lax
from jax.experimental import pallas as pl
from jax.experimental.pallas import tpu as pltpu, tpu_sc as plsc

D_MODEL = 64
SCALE = 8.0
VOCAB = 1000000

_info = plsc.get_sparse_core_info()
_NC, _NS, _L = _info.num_cores, _info.num_subcores, _info.num_lanes
_NW = _NC * _NS

_C = 256
_IDX_W = 128
_G = _C // _IDX_W
_NBUF = 4
_LA = 2

_BT = 12800
_K = 512000
_NBLK = _K // _BT
_TBLK_MAX = (VOCAB + _BT - 1) // _BT - 1


def _transpose_scale(table_t):
    def body(a_ref, b_ref, o_ref):
        o_ref[...] = jnp.concatenate(
            [a_ref[...].T, b_ref[...].T], axis=1) * SCALE

    return pl.pallas_call(
        body,
        grid=(_NBLK,),
        in_specs=[
            pl.BlockSpec((D_MODEL, _BT), lambda j: (0, j)),
            pl.BlockSpec(
                (D_MODEL, _BT),
                lambda j: (0, jnp.minimum(j + _NBLK, _TBLK_MAX)),
            ),
        ],
        out_specs=pl.BlockSpec((_BT, 128), lambda j: (j, 0)),
        out_shape=jax.ShapeDtypeStruct((_K, 128), jnp.float32),
    )(table_t, table_t)


def _make_lookup(B: int):
    assert B % (_NW * _C) == 0
    b_per_w = B // _NW
    n_chunks = b_per_w // _C
    assert n_chunks % _NBUF == 0 and n_chunks > _NBUF
    mesh = plsc.VectorSubcoreMesh(core_axis_name="c", subcore_axis_name="s")

    @functools.partial(
        pl.kernel,
        mesh=mesh,
        compiler_params=pltpu.CompilerParams(use_tc_tiling_on_sc=False),
        out_type=jax.ShapeDtypeStruct((B, 128), jnp.float32),
        scratch_types=[
            pltpu.VMEM((b_per_w,), jnp.int32),
            pltpu.VMEM((_NBUF, _C, D_MODEL), jnp.float32),
            pltpu.SemaphoreType.DMA((_NBUF,)),
            pltpu.SemaphoreType.DMA((_NBUF,)),
        ],
    )
    def lookup(idx_hbm, table_hbm, out_hbm, idx_v, rows_v, semg, semo):
        wid = lax.axis_index("s") * _NC + lax.axis_index("c")
        base = wid * b_per_w

        pltpu.sync_copy(idx_hbm.at[pl.ds(base, b_per_w)], idx_v)

        @plsc.parallel_loop(0, b_per_w // _L, unroll=8)
        def _remap(i):
            sl = pl.ds(i * _L, _L)
            v = idx_v[sl]
            idx_v[sl] = v * 2 - jnp.where(v >= _K, 2 * _K - 1, 0)

        def gather_descs(h, b):
            return [
                pltpu.make_async_copy(
                    table_hbm.at[idx_v.at[pl.ds(h * _C + j * _IDX_W, _IDX_W)]],
                    rows_v.at[b, pl.ds(j * _IDX_W, _IDX_W)],
                    semg.at[b],
                )
                for j in range(_G)
            ]

        def out_desc(g, b):
            return pltpu.make_async_copy(
                rows_v.at[b],
                out_hbm.at[pl.ds(base + g * _C, _C), pl.ds(0, D_MODEL)],
                semo.at[b],
            )

        def fire_gather(h, b):
            for d in gather_descs(h, b):
                d.start()

        for h in range(_LA):
            fire_gather(h, h % _NBUF)

        @pl.loop(0, n_chunks, step=_NBUF)
        def _chunk_group(go):
            for b in range(_NBUF):
                g = go + b
                for d in gather_descs(g, b):
                    d.wait()
                out_desc(g, b).start()
                bh = (b + _LA) % _NBUF

                @pl.when(jnp.logical_and(g + _LA < n_chunks,
                                         g + _LA - _NBUF >= 0))
                def _wait_old_writeback():
                    out_desc(g + _LA - _NBUF, bh).wait()

                @pl.when(g + _LA < n_chunks)
                def _fire_next():
                    fire_gather(g + _LA, bh)

        for k in range(_NBUF):
            g = n_chunks - _NBUF + k
            out_desc(g, g % _NBUF).wait()

    return lookup


_lookup_819200 = _make_lookup(4096 * 200)


def kernel(x, table):
    flat = x.reshape(-1).astype(jnp.int32)
    scaled_wide = _transpose_scale(table.T)
    scaled_rows = scaled_wide.reshape(2 * _K, D_MODEL)
    out = _lookup_819200(flat, scaled_rows)
    return out[:, :D_MODEL].reshape(x.shape + (D_MODEL,))


# speedup vs baseline: 1.2099x; 1.0025x over previous
import functools

import jax
import jax.numpy as jnp
from jax import lax
from jax.experimental import pallas as pl
from jax.experimental.pallas import tpu as pltpu, tpu_sc as plsc

D_MODEL = 64
SCALE = 8.0
VOCAB = 1000000

_info = plsc.get_sparse_core_info()
_NC, _NS, _L = _info.num_cores, _info.num_subcores, _info.num_lanes
_NW = _NC * _NS

_C = 256
_IDX_W = 128
_G = _C // _IDX_W
_NBUF = 4
_LA = 2

_BT = 16000
_K = 512000
_NBLK = _K // _BT
_TBLK_MAX = (VOCAB + _BT - 1) // _BT - 1


def _transpose_scale(table_t):
    def body(a_ref, b_ref, o_ref):
        o_ref[...] = jnp.concatenate(
            [a_ref[...].T, b_ref[...].T], axis=1) * SCALE

    return pl.pallas_call(
        body,
        grid=(_NBLK,),
        in_specs=[
            pl.BlockSpec((D_MODEL, _BT), lambda j: (0, j)),
            pl.BlockSpec(
                (D_MODEL, _BT),
                lambda j: (0, jnp.minimum(j + _NBLK, _TBLK_MAX)),
            ),
        ],
        out_specs=pl.BlockSpec((_BT, 128), lambda j: (j, 0)),
        out_shape=jax.ShapeDtypeStruct((_K, 128), jnp.float32),
    )(table_t, table_t)


def _make_lookup(B: int):
    assert B % (_NW * _C) == 0
    b_per_w = B // _NW
    n_chunks = b_per_w // _C
    assert n_chunks % _NBUF == 0 and n_chunks > _NBUF
    mesh = plsc.VectorSubcoreMesh(core_axis_name="c", subcore_axis_name="s")

    @functools.partial(
        pl.kernel,
        mesh=mesh,
        compiler_params=pltpu.CompilerParams(use_tc_tiling_on_sc=False),
        out_type=jax.ShapeDtypeStruct((B, 128), jnp.float32),
        scratch_types=[
            pltpu.VMEM((b_per_w,), jnp.int32),
            pltpu.VMEM((_NBUF, _C, D_MODEL), jnp.float32),
            pltpu.SemaphoreType.DMA((_NBUF,)),
            pltpu.SemaphoreType.DMA((_NBUF,)),
        ],
    )
    def lookup(idx_hbm, table_hbm, out_hbm, idx_v, rows_v, semg, semo):
        wid = lax.axis_index("s") * _NC + lax.axis_index("c")
        base = wid * b_per_w

        pltpu.sync_copy(idx_hbm.at[pl.ds(base, b_per_w)], idx_v)

        @plsc.parallel_loop(0, b_per_w // _L, unroll=8)
        def _remap(i):
            sl = pl.ds(i * _L, _L)
            v = idx_v[sl]
            idx_v[sl] = v * 2 - jnp.where(v >= _K, 2 * _K - 1, 0)

        def gather_descs(h, b):
            return [
                pltpu.make_async_copy(
                    table_hbm.at[idx_v.at[pl.ds(h * _C + j * _IDX_W, _IDX_W)]],
                    rows_v.at[b, pl.ds(j * _IDX_W, _IDX_W)],
                    semg.at[b],
                )
                for j in range(_G)
            ]

        def out_desc(g, b):
            return pltpu.make_async_copy(
                rows_v.at[b],
                out_hbm.at[pl.ds(base + g * _C, _C), pl.ds(0, D_MODEL)],
                semo.at[b],
            )

        def fire_gather(h, b):
            for d in gather_descs(h, b):
                d.start()

        for h in range(_LA):
            fire_gather(h, h % _NBUF)

        @pl.loop(0, n_chunks, step=_NBUF)
        def _chunk_group(go):
            for b in range(_NBUF):
                g = go + b
                for d in gather_descs(g, b):
                    d.wait()
                out_desc(g, b).start()
                bh = (b + _LA) % _NBUF

                @pl.when(jnp.logical_and(g + _LA < n_chunks,
                                         g + _LA - _NBUF >= 0))
                def _wait_old_writeback():
                    out_desc(g + _LA - _NBUF, bh).wait()

                @pl.when(g + _LA < n_chunks)
                def _fire_next():
                    fire_gather(g + _LA, bh)

        for k in range(_NBUF):
            g = n_chunks - _NBUF + k
            out_desc(g, g % _NBUF).wait()

    return lookup


_lookup_819200 = _make_lookup(4096 * 200)


def kernel(x, table):
    flat = x.reshape(-1).astype(jnp.int32)
    scaled_wide = _transpose_scale(table.T)
    scaled_rows = scaled_wide.reshape(2 * _K, D_MODEL)
    out = _lookup_819200(flat, scaled_rows)
    return out[:, :D_MODEL].reshape(x.shape + (D_MODEL,))


# speedup vs baseline: 1.2127x; 1.0023x over previous
import functools

import jax
import jax.numpy as jnp
from jax import lax
from jax.experimental import pallas as pl
from jax.experimental.pallas import tpu as pltpu, tpu_sc as plsc

D_MODEL = 64
SCALE = 8.0
VOCAB = 1000000

_info = plsc.get_sparse_core_info()
_NC, _NS, _L = _info.num_cores, _info.num_subcores, _info.num_lanes
_NW = _NC * _NS

_C = 256
_IDX_W = 256
_G = _C // _IDX_W
_NBUF = 4
_LA = 2

_BT = 16000
_K = 512000
_NBLK = _K // _BT
_TBLK_MAX = (VOCAB + _BT - 1) // _BT - 1


def _transpose_scale(table_t):
    def body(a_ref, b_ref, o_ref):
        o_ref[...] = jnp.concatenate(
            [a_ref[...].T, b_ref[...].T], axis=1) * SCALE

    return pl.pallas_call(
        body,
        grid=(_NBLK,),
        in_specs=[
            pl.BlockSpec((D_MODEL, _BT), lambda j: (0, j)),
            pl.BlockSpec(
                (D_MODEL, _BT),
                lambda j: (0, jnp.minimum(j + _NBLK, _TBLK_MAX)),
            ),
        ],
        out_specs=pl.BlockSpec((_BT, 128), lambda j: (j, 0)),
        out_shape=jax.ShapeDtypeStruct((_K, 128), jnp.float32),
    )(table_t, table_t)


def _make_lookup(B: int):
    assert B % (_NW * _C) == 0
    b_per_w = B // _NW
    n_chunks = b_per_w // _C
    assert n_chunks % _NBUF == 0 and n_chunks > _NBUF
    mesh = plsc.VectorSubcoreMesh(core_axis_name="c", subcore_axis_name="s")

    @functools.partial(
        pl.kernel,
        mesh=mesh,
        compiler_params=pltpu.CompilerParams(use_tc_tiling_on_sc=False),
        out_type=jax.ShapeDtypeStruct((B, 128), jnp.float32),
        scratch_types=[
            pltpu.VMEM((b_per_w,), jnp.int32),
            pltpu.VMEM((_NBUF, _C, D_MODEL), jnp.float32),
            pltpu.SemaphoreType.DMA((_NBUF,)),
            pltpu.SemaphoreType.DMA((_NBUF,)),
        ],
    )
    def lookup(idx_hbm, table_hbm, out_hbm, idx_v, rows_v, semg, semo):
        wid = lax.axis_index("s") * _NC + lax.axis_index("c")
        base = wid * b_per_w

        pltpu.sync_copy(idx_hbm.at[pl.ds(base, b_per_w)], idx_v)

        @plsc.parallel_loop(0, b_per_w // _L, unroll=8)
        def _remap(i):
            sl = pl.ds(i * _L, _L)
            v = idx_v[sl]
            idx_v[sl] = v * 2 - jnp.where(v >= _K, 2 * _K - 1, 0)

        def gather_descs(h, b):
            return [
                pltpu.make_async_copy(
                    table_hbm.at[idx_v.at[pl.ds(h * _C + j * _IDX_W, _IDX_W)]],
                    rows_v.at[b, pl.ds(j * _IDX_W, _IDX_W)],
                    semg.at[b],
                )
                for j in range(_G)
            ]

        def out_desc(g, b):
            return pltpu.make_async_copy(
                rows_v.at[b],
                out_hbm.at[pl.ds(base + g * _C, _C), pl.ds(0, D_MODEL)],
                semo.at[b],
            )

        def fire_gather(h, b):
            for d in gather_descs(h, b):
                d.start()

        for h in range(_LA):
            fire_gather(h, h % _NBUF)

        @pl.loop(0, n_chunks, step=_NBUF)
        def _chunk_group(go):
            for b in range(_NBUF):
                g = go + b
                for d in gather_descs(g, b):
                    d.wait()
                out_desc(g, b).start()
                bh = (b + _LA) % _NBUF

                @pl.when(jnp.logical_and(g + _LA < n_chunks,
                                         g + _LA - _NBUF >= 0))
                def _wait_old_writeback():
                    out_desc(g + _LA - _NBUF, bh).wait()

                @pl.when(g + _LA < n_chunks)
                def _fire_next():
                    fire_gather(g + _LA, bh)

        for k in range(_NBUF):
            g = n_chunks - _NBUF + k
            out_desc(g, g % _NBUF).wait()

    return lookup


_lookup_819200 = _make_lookup(4096 * 200)


def kernel(x, table):
    flat = x.reshape(-1).astype(jnp.int32)
    scaled_wide = _transpose_scale(table.T)
    scaled_rows = scaled_wide.reshape(2 * _K, D_MODEL)
    out = _lookup_819200(flat, scaled_rows)
    return out[:, :D_MODEL].reshape(x.shape + (D_MODEL,))
